# baseline (speedup 1.0000x reference)
import functools

import jax
import jax.numpy as jnp
from jax import lax
from jax.experimental import pallas as pl
from jax.experimental.pallas import tpu as pltpu
from jax.experimental.pallas import tpu_sc as plsc

B = 16384
F = 26
V = 100001
D = 32

NC = 2
NS = 16
NW = NC * NS

ROWS_PW = B * F // NW
IDX_ROW = 128
NCH = ROWS_PW // IDX_ROW
SUP = 13
NSUP = NCH // SUP
CH_ROWS = SUP * IDX_ROW


def _emb_body(tab_hbm, idx_hbm, out_hbm, idx_v, rows0, rows1, gsem, ssem):
  wid = lax.axis_index("s") * NC + lax.axis_index("c")
  base = wid * ROWS_PW

  pltpu.sync_copy(idx_hbm.at[wid], idx_v)

  viota = lax.iota(jnp.int32, 16)

  def fix_row(j, carry):
    row = idx_v.at[j]
    for k in range(IDX_ROW // 16):
      p0 = (j * IDX_ROW + k * 16) % F
      off = lax.rem(viota + p0, F) * V
      row[pl.ds(k * 16, 16)] = row[pl.ds(k * 16, 16)] + off
    return carry

  lax.fori_loop(0, NCH, fix_row, 0)

  bufs = (rows0, rows1)

  def fire(s, buf):
    for b in range(SUP):
      pltpu.async_copy(
          tab_hbm.at[idx_v.at[s * SUP + b]],
          buf.at[pl.ds(b * IDX_ROW, IDX_ROW)],
          gsem,
      )

  def drain_gathers(buf):
    for b in range(SUP):
      pltpu.make_async_copy(
          tab_hbm.at[idx_v.at[b]],
          buf.at[pl.ds(b * IDX_ROW, IDX_ROW)],
          gsem,
      ).wait()

  def scatter(s, buf):
    return pltpu.async_copy(
        buf, out_hbm.at[pl.ds(base + s * CH_ROWS, CH_ROWS)], ssem)

  fire(0, bufs[0])
  fire(1, bufs[1])
  for s in range(NSUP):
    buf = bufs[s % 2]
    drain_gathers(buf)
    sc = scatter(s, buf)
    if s + 2 < NSUP:
      sc.wait()
      fire(s + 2, buf)
    else:
      sc.wait()


@jax.jit
def kernel(indices, tables):
  tab_flat = tables.reshape(F * V, D)
  idx3 = indices.reshape(NW, NCH, IDX_ROW)
  mesh = plsc.VectorSubcoreMesh(
      core_axis_name="c", subcore_axis_name="s", num_cores=NC, num_subcores=NS)
  run = functools.partial(
      pl.kernel,
      out_type=jax.ShapeDtypeStruct((B * F, D), jnp.float32),
      mesh=mesh,
      scratch_types=[
          pltpu.VMEM((NCH, IDX_ROW), jnp.int32),
          pltpu.VMEM((CH_ROWS, D), jnp.float32),
          pltpu.VMEM((CH_ROWS, D), jnp.float32),
          pltpu.SemaphoreType.DMA,
          pltpu.SemaphoreType.DMA,
      ],
      compiler_params=pltpu.CompilerParams(use_tc_tiling_on_sc=False),
  )(_emb_body)
  out = run(tab_flat, idx3)
  return out.reshape(B, F, D)


# speedup vs baseline: 31.9334x; 31.9334x over previous
import functools

import jax
import jax.numpy as jnp
from jax import lax
from jax.experimental import pallas as pl
from jax.experimental.pallas import tpu as pltpu
from jax.experimental.pallas import tpu_sc as plsc

B = 16384
F = 26
V = 100001
D = 32

NC = 2
NS = 16
NW = NC * NS
VEC_PW = F * D // NW
HALF = B // 2


def _emb_body(tab_hbm, idx_hbm, out_hbm, vvec, idxv, outv, sem):
  wid = lax.axis_index("s") * NC + lax.axis_index("c")

  for k in range(VEC_PW):
    vid = wid * VEC_PW + k
    f = vid // D
    d = lax.rem(vid, D)
    pltpu.async_copy(tab_hbm.at[f, d], vvec, sem)
    pltpu.sync_copy(idx_hbm.at[f], idxv)
    pltpu.make_async_copy(tab_hbm.at[f, d], vvec, sem).wait()

    for half in range(2):
      def step(i, _, half=half):
        idx16 = idxv[pl.ds(half * HALF + i * 16, 16)]
        outv[pl.ds(i * 16, 16)] = plsc.load_gather(vvec, [idx16])
        return 0

      lax.fori_loop(0, HALF // 16, step, 0)
      pltpu.sync_copy(outv, out_hbm.at[f, d, pl.ds(half * HALF, HALF)])


@jax.jit
def kernel(indices, tables):
  tabT = jnp.transpose(tables, (0, 2, 1))
  idxT = jnp.transpose(indices, (1, 0))
  mesh = plsc.VectorSubcoreMesh(
      core_axis_name="c", subcore_axis_name="s", num_cores=NC, num_subcores=NS)
  run = functools.partial(
      pl.kernel,
      out_type=jax.ShapeDtypeStruct((F, D, B), jnp.float32),
      mesh=mesh,
      scratch_types=[
          pltpu.VMEM((V,), jnp.float32),
          pltpu.VMEM((B,), jnp.int32),
          pltpu.VMEM((HALF,), jnp.float32),
          pltpu.SemaphoreType.DMA,
      ],
      compiler_params=pltpu.CompilerParams(needs_layout_passes=False),
  )(_emb_body)
  outT = run(tabT, idxT)
  return jnp.transpose(outT, (2, 0, 1))
